# baseline (speedup 1.0000x reference)
import functools

import jax
import jax.numpy as jnp
from jax import lax
from jax.experimental import pallas as pl
from jax.experimental.pallas import tpu as pltpu
from jax.experimental.pallas import tpu_sc as plsc

N = 10000
E = 320000
D = 128
DEPTH = 3

NC = 2
NS = 16
NW = NC * NS

EPT = E // NW
CH = 80
NCHUNK = EPT // CH
EPTP = 10240
CH2 = 128
NCHA = EPTP // CH2
EPAD = NW * EPTP
NP = 10240
RPT = NP // NS
ZR = 80
NZ = RPT // ZR


def _sc_agg_body(hp, srcr, dstr, zrows_hbm, out,
                 src_v, dst_v, rows_v, acc_sh, sem):
    c = lax.axis_index("c")
    s = lax.axis_index("s")
    wid = s * NC + c
    r0 = s * RPT

    pltpu.sync_copy(srcr.at[wid], src_v)
    pltpu.sync_copy(dstr.at[wid], dst_v)

    pltpu.sync_copy(zrows_hbm, rows_v)
    for z in range(RPT // CH2):
        pltpu.sync_copy(rows_v, acc_sh.at[pl.ds(r0 + z * CH2, CH2), :])

    plsc.subcore_barrier()

    def _chunk(j, carry):
        pltpu.async_copy(hp.at[src_v.at[j]], rows_v, sem).wait()
        pltpu.sync_copy(rows_v, acc_sh.at[dst_v.at[j]], add=True)
        return carry
    lax.fori_loop(0, NCHA, _chunk, 0)

    plsc.subcore_barrier()

    for z in range(RPT // CH2):
        rs = pl.ds(r0 + z * CH2, CH2)
        pltpu.sync_copy(acc_sh.at[rs, :], rows_v)
        pltpu.sync_copy(rows_v, out.at[c, rs, :])


_sc_agg = pl.kernel(
    _sc_agg_body,
    out_type=[jax.ShapeDtypeStruct((NC, NP, D), jnp.float32)],
    mesh=plsc.VectorSubcoreMesh(core_axis_name="c", subcore_axis_name="s",
                                num_cores=NC, num_subcores=NS),
    scratch_types=[
        pltpu.VMEM((NCHA, CH2), jnp.int32),
        pltpu.VMEM((NCHA, CH2), jnp.int32),
        pltpu.VMEM((CH2, D), jnp.float32),
        pltpu.VMEM_SHARED((NP, D), jnp.float32),
        pltpu.SemaphoreType.DMA,
    ],
)


def _sc_cnt_body(dstr, ones_hbm, zeros_hbm, cnt_out, dst_v, ones_v, cz_v, cnt_sh):
    c = lax.axis_index("c")
    s = lax.axis_index("s")
    wid = s * NC + c

    pltpu.sync_copy(dstr.at[wid], dst_v)
    pltpu.sync_copy(ones_hbm, ones_v)
    pltpu.sync_copy(zeros_hbm, cz_v)

    r0 = s * RPT
    for z in range(NZ):
        pltpu.sync_copy(cz_v, cnt_sh.at[pl.ds(r0 + z * ZR, ZR), :])

    plsc.subcore_barrier()

    def _chunk(j, carry):
        pltpu.sync_copy(ones_v, cnt_sh.at[dst_v.at[j]], add=True)
        return carry
    lax.fori_loop(0, NCHUNK, _chunk, 0)

    plsc.subcore_barrier()

    for z in range(NZ):
        rs = pl.ds(r0 + z * ZR, ZR)
        pltpu.sync_copy(cnt_sh.at[rs, :], cz_v)
        pltpu.sync_copy(cz_v, cnt_out.at[c, rs, :])


_sc_cnt = pl.kernel(
    _sc_cnt_body,
    out_type=[jax.ShapeDtypeStruct((NC, NP, D), jnp.float32)],
    mesh=plsc.VectorSubcoreMesh(core_axis_name="c", subcore_axis_name="s",
                                num_cores=NC, num_subcores=NS),
    scratch_types=[
        pltpu.VMEM((NCHUNK, CH), jnp.int32),
        pltpu.VMEM((CH, D), jnp.float32),
        pltpu.VMEM((ZR, D), jnp.float32),
        pltpu.VMEM_SHARED((NP, D), jnp.float32),
    ],
)

def _k0_body(x_ref, pos_ref, freq_ref, wpt_ref, bp_ref, h_ref, hp_ref):
    ang = pos_ref[...] * freq_ref[...]
    col = lax.broadcasted_iota(jnp.int32, (1, D), 1)
    pe = jnp.where(col < D // 2, jnp.sin(ang), jnp.cos(ang))
    h = x_ref[...] + pe
    h_ref[...] = h
    hp_ref[...] = jnp.maximum(
        jnp.dot(h, wpt_ref[...], preferred_element_type=jnp.float32)
        + bp_ref[...], 0.0)


_k0 = pl.pallas_call(
    _k0_body,
    out_shape=[jax.ShapeDtypeStruct((N, D), jnp.float32),
               jax.ShapeDtypeStruct((N, D), jnp.float32)],
)


def _norm_block(acc_ref, cnt_ref, h_ref, wlt, bl, wrt, g, b):
    deg = cnt_ref[0, :N, 0:1] + cnt_ref[1, :N, 0:1]
    agg = (acc_ref[0, :N, :] + acc_ref[1, :N, :]) / jnp.maximum(deg, 1.0)
    y = (jnp.dot(agg, wlt[...], preferred_element_type=jnp.float32) + bl[...]
         + jnp.dot(h_ref[...], wrt[...], preferred_element_type=jnp.float32))
    mean = jnp.mean(y)
    xc = y - mean
    std = jnp.sqrt(jnp.mean(xc * xc))
    hn = (xc / (std + 1e-5)) * g[...] + b[...]
    return jnp.where(hn > 0, hn, 0.2 * hn)


def _layer_body(acc_ref, cnt_ref, h_ref, wlt, bl, wrt, g, b, wnt, bn,
                hn_ref, hp_ref):
    hn = _norm_block(acc_ref, cnt_ref, h_ref, wlt, bl, wrt, g, b)
    hn_ref[...] = hn
    hp_ref[...] = jnp.maximum(
        jnp.dot(hn, wnt[...], preferred_element_type=jnp.float32)
        + bn[...], 0.0)


_k_layer = pl.pallas_call(
    _layer_body,
    out_shape=[jax.ShapeDtypeStruct((N, D), jnp.float32),
               jax.ShapeDtypeStruct((N, D), jnp.float32)],
)


def _final_body(acc_ref, cnt_ref, h_ref, wlt, bl, wrt, g, b, wft, bf, x_ref,
                out_ref):
    hn = _norm_block(acc_ref, cnt_ref, h_ref, wlt, bl, wrt, g, b)
    out_ref[...] = (x_ref[...]
                    + jnp.dot(hn, wft[...], preferred_element_type=jnp.float32)
                    + bf[...])


_k_final = pl.pallas_call(
    _final_body,
    out_shape=jax.ShapeDtypeStruct((N, D), jnp.float32),
)


def kernel(x, pos, Wp, bp, Wl, bl, Wr, gamma, beta, Wf, bf, edge_index, batch):
    pad_n = EPAD - E
    src = jnp.concatenate(
        [edge_index[0], jnp.zeros((pad_n,), jnp.int32)]).reshape(NW, NCHA, CH2)
    pad_dst = N + (jnp.arange(pad_n, dtype=jnp.int32) % (NP - N))
    dst = jnp.concatenate([edge_index[1], pad_dst]).reshape(NW, NCHA, CH2)
    dst_c = edge_index[1].reshape(NW, NCHUNK, CH)
    dst_c = edge_index[1].reshape(NW, NCHUNK, CH)

    freq = jnp.power(jnp.float32(1e-4),
                     jnp.linspace(0.0, 1.0, D // 2, dtype=jnp.float32))
    freq2 = jnp.concatenate([freq, freq]).reshape(1, D)
    pos2 = pos.reshape(N, 1)

    r = lambda v: v.reshape(1, D)

    h, hp = _k0(x, pos2, freq2, Wp[0].T, r(bp[0]))

    zrows = jnp.zeros((CH2, D), jnp.float32)
    ones16 = jnp.ones((CH, D), jnp.float32)
    zeros16 = jnp.zeros((ZR, D), jnp.float32)

    (cnt,) = _sc_cnt(dst_c, ones16, zeros16)
    (acc,) = _sc_agg(hp, src, dst, zrows)
    h, hp = _k_layer(acc, cnt, h, Wl[0].T, r(bl[0]), Wr[0].T,
                     r(gamma[0]), r(beta[0]), Wp[1].T, r(bp[1]))

    (acc,) = _sc_agg(hp, src, dst, zrows)
    h, hp = _k_layer(acc, cnt, h, Wl[1].T, r(bl[1]), Wr[1].T,
                     r(gamma[1]), r(beta[1]), Wp[2].T, r(bp[2]))

    (acc,) = _sc_agg(hp, src, dst, zrows)
    out = _k_final(acc, cnt, h, Wl[2].T, r(bl[2]), Wr[2].T,
                   r(gamma[2]), r(beta[2]), Wf.T, r(bf), x)
    return out


# speedup vs baseline: 2.5782x; 2.5782x over previous
import functools

import jax
import jax.numpy as jnp
from jax import lax
from jax.experimental import pallas as pl
from jax.experimental.pallas import tpu as pltpu
from jax.experimental.pallas import tpu_sc as plsc

N = 10000
E = 320000
D = 128
DEPTH = 3

NC = 2
NS = 16
NW = NC * NS

EPT = E // NW
CH = 80
NCHUNK = EPT // CH
EPTP = 10240
CH2 = 128
NCHA = EPTP // CH2
EPAD = NW * EPTP
NP = 10240
RPT = NP // NS
ZR = 80
NZ = RPT // ZR


def _sc_agg_body(hp, srcr, dstr, zrows_hbm, out,
                 src_v, dst_v, rows_v, acc_sh, sem):
    c = lax.axis_index("c")
    s = lax.axis_index("s")
    wid = s * NC + c
    r0 = s * RPT

    pltpu.sync_copy(srcr.at[wid], src_v)
    pltpu.sync_copy(dstr.at[wid], dst_v)

    pltpu.sync_copy(zrows_hbm, rows_v)
    for z in range(RPT // CH2):
        pltpu.sync_copy(rows_v, acc_sh.at[pl.ds(r0 + z * CH2, CH2), :])

    plsc.subcore_barrier()

    def _chunk(j, carry):
        pltpu.async_copy(hp.at[src_v.at[j]], rows_v, sem).wait()
        pltpu.sync_copy(rows_v, acc_sh.at[dst_v.at[j]], add=True)
        return carry
    lax.fori_loop(0, NCHA, _chunk, 0)

    plsc.subcore_barrier()

    for z in range(RPT // CH2):
        rs = pl.ds(r0 + z * CH2, CH2)
        pltpu.sync_copy(acc_sh.at[rs, :], rows_v)
        pltpu.sync_copy(rows_v, out.at[c, rs, :])


_sc_agg = pl.kernel(
    _sc_agg_body,
    out_type=[jax.ShapeDtypeStruct((NC, NP, D), jnp.float32)],
    mesh=plsc.VectorSubcoreMesh(core_axis_name="c", subcore_axis_name="s",
                                num_cores=NC, num_subcores=NS),
    scratch_types=[
        pltpu.VMEM((NCHA, CH2), jnp.int32),
        pltpu.VMEM((NCHA, CH2), jnp.int32),
        pltpu.VMEM((CH2, D), jnp.float32),
        pltpu.VMEM_SHARED((NP, D), jnp.float32),
        pltpu.SemaphoreType.DMA,
    ],
)


def _sc_cnt_body(dstr, ones_hbm, zeros_hbm, cnt_out, dst_v, ones_v, cz_v, cnt_sh):
    c = lax.axis_index("c")
    s = lax.axis_index("s")
    wid = s * NC + c

    pltpu.sync_copy(dstr.at[wid], dst_v)
    pltpu.sync_copy(ones_hbm, ones_v)
    pltpu.sync_copy(zeros_hbm, cz_v)

    r0 = s * RPT
    for z in range(NZ):
        pltpu.sync_copy(cz_v, cnt_sh.at[pl.ds(r0 + z * ZR, ZR), :])

    plsc.subcore_barrier()

    def _chunk(j, carry):
        pltpu.sync_copy(ones_v, cnt_sh.at[dst_v.at[j]], add=True)
        return carry
    lax.fori_loop(0, NCHUNK, _chunk, 0)

    plsc.subcore_barrier()

    for z in range(NZ):
        rs = pl.ds(r0 + z * ZR, ZR)
        pltpu.sync_copy(cnt_sh.at[rs, :], cz_v)
        pltpu.sync_copy(cz_v, cnt_out.at[c, rs, :])


_sc_cnt = pl.kernel(
    _sc_cnt_body,
    out_type=[jax.ShapeDtypeStruct((NC, NP, D), jnp.float32)],
    mesh=plsc.VectorSubcoreMesh(core_axis_name="c", subcore_axis_name="s",
                                num_cores=NC, num_subcores=NS),
    scratch_types=[
        pltpu.VMEM((NCHUNK, CH), jnp.int32),
        pltpu.VMEM((CH, D), jnp.float32),
        pltpu.VMEM((ZR, D), jnp.float32),
        pltpu.VMEM_SHARED((NP, D), jnp.float32),
    ],
)

def _k0_body(x_ref, pos_ref, freq_ref, wpt_ref, bp_ref, h_ref, hp_ref):
    ang = pos_ref[...] * freq_ref[...]
    col = lax.broadcasted_iota(jnp.int32, (1, D), 1)
    pe = jnp.where(col < D // 2, jnp.sin(ang), jnp.cos(ang))
    h = x_ref[...] + pe
    h_ref[...] = h
    hp_ref[...] = jnp.maximum(
        jnp.dot(h, wpt_ref[...], preferred_element_type=jnp.float32)
        + bp_ref[...], 0.0)


_k0 = pl.pallas_call(
    _k0_body,
    out_shape=[jax.ShapeDtypeStruct((N, D), jnp.float32),
               jax.ShapeDtypeStruct((N, D), jnp.float32)],
)


def _norm_block(acc_ref, cnt_ref, h_ref, wlt, bl, wrt, g, b):
    deg = cnt_ref[0, :N, 0:1] + cnt_ref[1, :N, 0:1]
    agg = (acc_ref[0, :N, :] + acc_ref[1, :N, :]) / jnp.maximum(deg, 1.0)
    y = (jnp.dot(agg, wlt[...], preferred_element_type=jnp.float32) + bl[...]
         + jnp.dot(h_ref[...], wrt[...], preferred_element_type=jnp.float32))
    mean = jnp.mean(y)
    xc = y - mean
    std = jnp.sqrt(jnp.mean(xc * xc))
    hn = (xc / (std + 1e-5)) * g[...] + b[...]
    return jnp.where(hn > 0, hn, 0.2 * hn)


def _layer_body(acc_ref, cnt_ref, h_ref, wlt, bl, wrt, g, b, wnt, bn,
                hn_ref, hp_ref):
    hn = _norm_block(acc_ref, cnt_ref, h_ref, wlt, bl, wrt, g, b)
    hn_ref[...] = hn
    hp_ref[...] = jnp.maximum(
        jnp.dot(hn, wnt[...], preferred_element_type=jnp.float32)
        + bn[...], 0.0)


_k_layer = pl.pallas_call(
    _layer_body,
    out_shape=[jax.ShapeDtypeStruct((N, D), jnp.float32),
               jax.ShapeDtypeStruct((N, D), jnp.float32)],
)


def _final_body(acc_ref, cnt_ref, h_ref, wlt, bl, wrt, g, b, wft, bf, x_ref,
                out_ref):
    hn = _norm_block(acc_ref, cnt_ref, h_ref, wlt, bl, wrt, g, b)
    out_ref[...] = (x_ref[...]
                    + jnp.dot(hn, wft[...], preferred_element_type=jnp.float32)
                    + bf[...])


_k_final = pl.pallas_call(
    _final_body,
    out_shape=jax.ShapeDtypeStruct((N, D), jnp.float32),
)


def kernel(x, pos, Wp, bp, Wl, bl, Wr, gamma, beta, Wf, bf, edge_index, batch):
    ppt = EPTP - EPT
    pad_src = jnp.broadcast_to(jnp.arange(ppt, dtype=jnp.int32), (NW, ppt))
    pad_dst = jnp.broadcast_to(N + jnp.arange(ppt, dtype=jnp.int32), (NW, ppt))
    src = jnp.concatenate(
        [edge_index[0].reshape(NW, EPT), pad_src], axis=1).reshape(NW, NCHA, CH2)
    dst = jnp.concatenate(
        [edge_index[1].reshape(NW, EPT), pad_dst], axis=1).reshape(NW, NCHA, CH2)
    dst_c = edge_index[1].reshape(NW, NCHUNK, CH)
    dst_c = edge_index[1].reshape(NW, NCHUNK, CH)

    freq = jnp.power(jnp.float32(1e-4),
                     jnp.linspace(0.0, 1.0, D // 2, dtype=jnp.float32))
    freq2 = jnp.concatenate([freq, freq]).reshape(1, D)
    pos2 = pos.reshape(N, 1)

    r = lambda v: v.reshape(1, D)

    h, hp = _k0(x, pos2, freq2, Wp[0].T, r(bp[0]))

    zrows = jnp.zeros((CH2, D), jnp.float32)
    ones16 = jnp.ones((CH, D), jnp.float32)
    zeros16 = jnp.zeros((ZR, D), jnp.float32)

    (cnt,) = _sc_cnt(dst_c, ones16, zeros16)
    (acc,) = _sc_agg(hp, src, dst, zrows)
    h, hp = _k_layer(acc, cnt, h, Wl[0].T, r(bl[0]), Wr[0].T,
                     r(gamma[0]), r(beta[0]), Wp[1].T, r(bp[1]))

    (acc,) = _sc_agg(hp, src, dst, zrows)
    h, hp = _k_layer(acc, cnt, h, Wl[1].T, r(bl[1]), Wr[1].T,
                     r(gamma[1]), r(beta[1]), Wp[2].T, r(bp[2]))

    (acc,) = _sc_agg(hp, src, dst, zrows)
    out = _k_final(acc, cnt, h, Wl[2].T, r(bl[2]), Wr[2].T,
                   r(gamma[2]), r(beta[2]), Wf.T, r(bf), x)
    return out


# speedup vs baseline: 2.5832x; 1.0019x over previous
import functools

import jax
import jax.numpy as jnp
from jax import lax
from jax.experimental import pallas as pl
from jax.experimental.pallas import tpu as pltpu
from jax.experimental.pallas import tpu_sc as plsc

N = 10000
E = 320000
D = 128
DEPTH = 3

NC = 2
NS = 16
NW = NC * NS

EPT = E // NW
CH = 80
NCHUNK = EPT // CH
EPTP = 10240
CH2 = 128
NCHA = EPTP // CH2
EPAD = NW * EPTP
NP = 10240
RPT = NP // NS
ZR = 80
NZ = RPT // ZR


def _sc_agg_body(hp, srcr, dstr, zrows_hbm, out,
                 src_v, dst_v, rows_v, acc_sh, sem):
    c = lax.axis_index("c")
    s = lax.axis_index("s")
    wid = s * NC + c
    r0 = s * RPT

    pltpu.sync_copy(srcr.at[wid], src_v)
    pltpu.sync_copy(dstr.at[wid], dst_v)

    pltpu.sync_copy(zrows_hbm, rows_v)
    for z in range(RPT // CH2):
        pltpu.sync_copy(rows_v, acc_sh.at[pl.ds(r0 + z * CH2, CH2), :])

    plsc.subcore_barrier()

    def _chunk(j, carry):
        pltpu.async_copy(hp.at[src_v.at[j]], rows_v, sem).wait()
        pltpu.sync_copy(rows_v, acc_sh.at[dst_v.at[j]], add=True)
        return carry
    lax.fori_loop(0, NCHA, _chunk, 0)

    plsc.subcore_barrier()

    for z in range(RPT // CH2):
        rs = pl.ds(r0 + z * CH2, CH2)
        pltpu.sync_copy(acc_sh.at[rs, :], rows_v)
        pltpu.sync_copy(rows_v, out.at[c, rs, :])


_sc_agg = pl.kernel(
    _sc_agg_body,
    out_type=[jax.ShapeDtypeStruct((NC, NP, D), jnp.float32)],
    mesh=plsc.VectorSubcoreMesh(core_axis_name="c", subcore_axis_name="s",
                                num_cores=NC, num_subcores=NS),
    scratch_types=[
        pltpu.VMEM((NCHA, CH2), jnp.int32),
        pltpu.VMEM((NCHA, CH2), jnp.int32),
        pltpu.VMEM((CH2, D), jnp.float32),
        pltpu.VMEM_SHARED((NP, D), jnp.float32),
        pltpu.SemaphoreType.DMA,
    ],
)


def _sc_cnt_body(dstr, ones_hbm, zeros_hbm, cnt_out, dst_v, ones_v, cz_v, cnt_sh):
    c = lax.axis_index("c")
    s = lax.axis_index("s")
    wid = s * NC + c

    pltpu.sync_copy(dstr.at[wid], dst_v)
    pltpu.sync_copy(ones_hbm, ones_v)
    pltpu.sync_copy(zeros_hbm, cz_v)

    r0 = s * RPT
    for z in range(RPT // CH2):
        pltpu.sync_copy(cz_v, cnt_sh.at[pl.ds(r0 + z * CH2, CH2), :])

    plsc.subcore_barrier()

    def _chunk(j, carry):
        pltpu.sync_copy(ones_v, cnt_sh.at[dst_v.at[j]], add=True)
        return carry
    lax.fori_loop(0, NCHA, _chunk, 0)

    plsc.subcore_barrier()

    for z in range(RPT // CH2):
        rs = pl.ds(r0 + z * CH2, CH2)
        pltpu.sync_copy(cnt_sh.at[rs, :], cz_v)
        pltpu.sync_copy(cz_v, cnt_out.at[c, rs, :])


_sc_cnt = pl.kernel(
    _sc_cnt_body,
    out_type=[jax.ShapeDtypeStruct((NC, NP, D), jnp.float32)],
    mesh=plsc.VectorSubcoreMesh(core_axis_name="c", subcore_axis_name="s",
                                num_cores=NC, num_subcores=NS),
    scratch_types=[
        pltpu.VMEM((NCHA, CH2), jnp.int32),
        pltpu.VMEM((CH2, D), jnp.float32),
        pltpu.VMEM((CH2, D), jnp.float32),
        pltpu.VMEM_SHARED((NP, D), jnp.float32),
    ],
)

def _k0_body(x_ref, pos_ref, freq_ref, wpt_ref, bp_ref, h_ref, hp_ref):
    ang = pos_ref[...] * freq_ref[...]
    col = lax.broadcasted_iota(jnp.int32, (1, D), 1)
    pe = jnp.where(col < D // 2, jnp.sin(ang), jnp.cos(ang))
    h = x_ref[...] + pe
    h_ref[...] = h
    hp_ref[...] = jnp.maximum(
        jnp.dot(h, wpt_ref[...], preferred_element_type=jnp.float32)
        + bp_ref[...], 0.0)


_k0 = pl.pallas_call(
    _k0_body,
    out_shape=[jax.ShapeDtypeStruct((N, D), jnp.float32),
               jax.ShapeDtypeStruct((N, D), jnp.float32)],
)


def _norm_block(acc_ref, cnt_ref, h_ref, wlt, bl, wrt, g, b):
    deg = cnt_ref[0, :N, 0:1] + cnt_ref[1, :N, 0:1]
    agg = (acc_ref[0, :N, :] + acc_ref[1, :N, :]) / jnp.maximum(deg, 1.0)
    y = (jnp.dot(agg, wlt[...], preferred_element_type=jnp.float32) + bl[...]
         + jnp.dot(h_ref[...], wrt[...], preferred_element_type=jnp.float32))
    mean = jnp.mean(y)
    xc = y - mean
    std = jnp.sqrt(jnp.mean(xc * xc))
    hn = (xc / (std + 1e-5)) * g[...] + b[...]
    return jnp.where(hn > 0, hn, 0.2 * hn)


def _layer_body(acc_ref, cnt_ref, h_ref, wlt, bl, wrt, g, b, wnt, bn,
                hn_ref, hp_ref):
    hn = _norm_block(acc_ref, cnt_ref, h_ref, wlt, bl, wrt, g, b)
    hn_ref[...] = hn
    hp_ref[...] = jnp.maximum(
        jnp.dot(hn, wnt[...], preferred_element_type=jnp.float32)
        + bn[...], 0.0)


_k_layer = pl.pallas_call(
    _layer_body,
    out_shape=[jax.ShapeDtypeStruct((N, D), jnp.float32),
               jax.ShapeDtypeStruct((N, D), jnp.float32)],
)


def _final_body(acc_ref, cnt_ref, h_ref, wlt, bl, wrt, g, b, wft, bf, x_ref,
                out_ref):
    hn = _norm_block(acc_ref, cnt_ref, h_ref, wlt, bl, wrt, g, b)
    out_ref[...] = (x_ref[...]
                    + jnp.dot(hn, wft[...], preferred_element_type=jnp.float32)
                    + bf[...])


_k_final = pl.pallas_call(
    _final_body,
    out_shape=jax.ShapeDtypeStruct((N, D), jnp.float32),
)


def kernel(x, pos, Wp, bp, Wl, bl, Wr, gamma, beta, Wf, bf, edge_index, batch):
    ppt = EPTP - EPT
    pad_src = jnp.broadcast_to(jnp.arange(ppt, dtype=jnp.int32), (NW, ppt))
    pad_dst = jnp.broadcast_to(N + jnp.arange(ppt, dtype=jnp.int32), (NW, ppt))
    src = jnp.concatenate(
        [edge_index[0].reshape(NW, EPT), pad_src], axis=1).reshape(NW, NCHA, CH2)
    dst = jnp.concatenate(
        [edge_index[1].reshape(NW, EPT), pad_dst], axis=1).reshape(NW, NCHA, CH2)

    freq = jnp.power(jnp.float32(1e-4),
                     jnp.linspace(0.0, 1.0, D // 2, dtype=jnp.float32))
    freq2 = jnp.concatenate([freq, freq]).reshape(1, D)
    pos2 = pos.reshape(N, 1)

    r = lambda v: v.reshape(1, D)

    h, hp = _k0(x, pos2, freq2, Wp[0].T, r(bp[0]))

    zrows = jnp.zeros((CH2, D), jnp.float32)
    ones_rows = jnp.ones((CH2, D), jnp.float32)

    (cnt,) = _sc_cnt(dst, ones_rows, zrows)
    (acc,) = _sc_agg(hp, src, dst, zrows)
    h, hp = _k_layer(acc, cnt, h, Wl[0].T, r(bl[0]), Wr[0].T,
                     r(gamma[0]), r(beta[0]), Wp[1].T, r(bp[1]))

    (acc,) = _sc_agg(hp, src, dst, zrows)
    h, hp = _k_layer(acc, cnt, h, Wl[1].T, r(bl[1]), Wr[1].T,
                     r(gamma[1]), r(beta[1]), Wp[2].T, r(bp[2]))

    (acc,) = _sc_agg(hp, src, dst, zrows)
    out = _k_final(acc, cnt, h, Wl[2].T, r(bl[2]), Wr[2].T,
                   r(gamma[2]), r(beta[2]), Wf.T, r(bf), x)
    return out
